# baseline (speedup 1.0000x reference)
import functools

import jax
import jax.numpy as jnp
from jax import lax
from jax.experimental import pallas as pl
from jax.experimental.pallas import tpu as pltpu
from jax.experimental.pallas import tpu_sc as plsc

_NEG = -1e30
_CHUNK = 16384
_UNROLL = 8


def _sc_bag_max(B, N):
    nchunk = N // _CHUNK
    mesh = plsc.VectorSubcoreMesh(core_axis_name="c", subcore_axis_name="s")
    ncores, nsub = 2, 16
    nw = ncores * nsub
    bpw = B // nw

    @functools.partial(
        pl.kernel,
        mesh=mesh,
        out_type=jax.ShapeDtypeStruct((B, 16), jnp.float32),
        scratch_types=[
            pltpu.VMEM((2, _CHUNK), jnp.float32),
            pltpu.VMEM((2, _CHUNK), jnp.int32),
            pltpu.VMEM((16,), jnp.int32),
            pltpu.VMEM((16,), jnp.float32),
            pltpu.SemaphoreType.DMA((2,)),
            pltpu.SemaphoreType.DMA((2,)),
        ],
    )
    def k(x_hbm, z_hbm, cats_hbm, out_hbm, xbuf, zbuf, cats_v, res_v, xsem, zsem):
        wid = lax.axis_index("s") * ncores + lax.axis_index("c")

        def start(b, c, slot):
            off = c * _CHUNK
            pltpu.async_copy(x_hbm.at[b, pl.ds(off, _CHUNK)], xbuf.at[slot],
                             xsem.at[slot])
            pltpu.async_copy(z_hbm.at[b, pl.ds(off, _CHUNK)], zbuf.at[slot],
                             zsem.at[slot])

        def wait(b, slot):
            pltpu.make_async_copy(x_hbm.at[0, pl.ds(0, _CHUNK)], xbuf.at[slot],
                                  xsem.at[slot]).wait()
            pltpu.make_async_copy(z_hbm.at[0, pl.ds(0, _CHUNK)], zbuf.at[slot],
                                  zsem.at[slot]).wait()

        total = bpw * nchunk
        start(wid * bpw, 0, 0)
        start(wid * bpw, 1, 1)

        for j in range(bpw):
            b = wid * bpw + j
            pltpu.sync_copy(cats_hbm.at[b], cats_v)
            cat = cats_v[...]
            acc = jnp.full((16,), _NEG, dtype=jnp.float32)
            for c in range(nchunk):
                g = j * nchunk + c
                slot = g % 2
                wait(b, slot)

                def inner(i, acc):
                    for u in range(_UNROLL):
                        base = i * (16 * _UNROLL) + u * 16
                        xv = xbuf[slot, pl.ds(base, 16)]
                        zv = zbuf[slot, pl.ds(base, 16)]
                        m = zv == cat
                        acc = jnp.where(m, jnp.maximum(acc, xv), acc)
                    return acc

                acc = lax.fori_loop(0, _CHUNK // (16 * _UNROLL), inner, acc)

                ng = g + 2
                if ng < total:
                    start(wid * bpw + ng // nchunk, ng % nchunk, slot)
            res_v[...] = acc
            pltpu.sync_copy(res_v, out_hbm.at[b])

    return k


def _bce_body(max_ref, cats_ref, labels_ref, out_ref):
    x = jnp.max(max_ref[:, 0, :], axis=1, keepdims=True)
    c = cats_ref[:, 0, :]
    y = labels_ref[:, 0, :]
    valid = (c > 0) & (x > -9e29)
    r = jnp.where(valid, x, 0.0)
    per = jnp.maximum(r, 0.0) - r * y + jnp.log1p(jnp.exp(-jnp.abs(r)))
    out_ref[0, 0] = jnp.sum(per) / per.shape[0]


def kernel(pixel_logits, zone_patches, cats, labels):
    B, _, H, W = pixel_logits.shape
    N = H * W
    logits = pixel_logits.reshape(B, N)
    zones = zone_patches.reshape(B, N)

    cats16 = jnp.broadcast_to(cats[:, None], (B, 16))
    bag_max = _sc_bag_max(B, N)(logits, zones, cats16)

    loss = pl.pallas_call(
        _bce_body,
        in_specs=[
            pl.BlockSpec((B, 1, 16), lambda: (0, 0, 0)),
            pl.BlockSpec((B, 1, 1), lambda: (0, 0, 0)),
            pl.BlockSpec((B, 1, 1), lambda: (0, 0, 0)),
        ],
        out_specs=pl.BlockSpec(memory_space=pltpu.SMEM),
        out_shape=jax.ShapeDtypeStruct((1, 1), jnp.float32),
    )(bag_max.reshape(B, 1, 16), cats.reshape(B, 1, 1), labels.reshape(B, 1, 1))

    return loss[0, 0]


# speedup vs baseline: 1.1579x; 1.1579x over previous
import functools

import jax
import jax.numpy as jnp
from jax import lax
from jax.experimental import pallas as pl
from jax.experimental.pallas import tpu as pltpu
from jax.experimental.pallas import tpu_sc as plsc

_NEG = -1e30
_NSLOT = 4
_CHUNK = 16384
_UNROLL = 8
_SC_B = 16


def _sc_bag_max(B, N):
    nchunk = N // _CHUNK
    mesh = plsc.VectorSubcoreMesh(core_axis_name="c", subcore_axis_name="s")
    ncores, nsub = 2, 16
    nw = ncores * nsub

    @functools.partial(
        pl.kernel,
        mesh=mesh,
        out_type=jax.ShapeDtypeStruct((nw, 16), jnp.float32),
        scratch_types=[
            pltpu.VMEM((2, _CHUNK), jnp.float32),
            pltpu.VMEM((2, _CHUNK), jnp.int32),
            pltpu.VMEM((16,), jnp.int32),
            pltpu.VMEM((16,), jnp.float32),
            pltpu.SemaphoreType.DMA((2,)),
            pltpu.SemaphoreType.DMA((2,)),
        ],
    )
    def k(x_hbm, z_hbm, cats_hbm, out_hbm, xbuf, zbuf, cats_v, res_v, xsem, zsem):
        wid = lax.axis_index("s") * ncores + lax.axis_index("c")
        total = B * nchunk
        cpw = total // nw
        base_g = wid * cpw
        my_b = base_g // nchunk

        def start(g, slot):
            b = g // nchunk
            off = (g % nchunk) * _CHUNK
            pltpu.async_copy(x_hbm.at[b, pl.ds(off, _CHUNK)], xbuf.at[slot],
                             xsem.at[slot])
            pltpu.async_copy(z_hbm.at[b, pl.ds(off, _CHUNK)], zbuf.at[slot],
                             zsem.at[slot])

        def wait(slot):
            pltpu.make_async_copy(x_hbm.at[0, pl.ds(0, _CHUNK)], xbuf.at[slot],
                                  xsem.at[slot]).wait()
            pltpu.make_async_copy(z_hbm.at[0, pl.ds(0, _CHUNK)], zbuf.at[slot],
                                  zsem.at[slot]).wait()

        start(base_g, 0)
        if cpw > 1:
            start(base_g + 1, 1)
        pltpu.sync_copy(cats_hbm.at[my_b], cats_v)
        cat = cats_v[...]

        acc = jnp.full((16,), _NEG, dtype=jnp.float32)
        for c in range(cpw):
            slot = c % 2
            wait(slot)

            def inner(i, acc):
                for u in range(_UNROLL):
                    base = i * (16 * _UNROLL) + u * 16
                    xv = xbuf[slot, pl.ds(base, 16)]
                    zv = zbuf[slot, pl.ds(base, 16)]
                    acc = jnp.where(zv == cat, jnp.maximum(acc, xv), acc)
                return acc

            acc = lax.fori_loop(0, _CHUNK // (16 * _UNROLL), inner, acc)

            if c + 2 < cpw:
                start(base_g + c + 2, slot)
        res_v[...] = acc
        pltpu.sync_copy(res_v, out_hbm.at[wid, :])

    return k


def _tc_body(cats_ref, labels_ref, x_hbm, z_hbm, out_ref, xbuf, zbuf, acc_ref,
             xsem, zsem):
    B = x_hbm.shape[0]

    def start(b, slot):
        pltpu.make_async_copy(x_hbm.at[b], xbuf.at[slot], xsem.at[slot]).start()
        pltpu.make_async_copy(z_hbm.at[b], zbuf.at[slot], zsem.at[slot]).start()

    for b in range(_NSLOT):
        start(b, b)

    def step(b, loss_sum):
        slot = jax.lax.rem(b, _NSLOT)
        pltpu.make_async_copy(x_hbm.at[0], xbuf.at[slot], xsem.at[slot]).wait()
        pltpu.make_async_copy(z_hbm.at[0], zbuf.at[slot], zsem.at[slot]).wait()
        x = xbuf[slot]
        z = zbuf[slot]
        cat = cats_ref[b]
        part = jnp.max(jnp.where(z == cat, x, _NEG))

        @pl.when(b + _NSLOT < B)
        def _next():
            start(b + _NSLOT, slot)

        valid = (cat > 0) & (part > -9e29)
        r = jnp.where(valid, part, 0.0)
        y = labels_ref[b]
        per = jnp.maximum(r, 0.0) - r * y + jnp.log1p(jnp.exp(-jnp.abs(r)))
        return loss_sum + per

    loss_sum = jax.lax.fori_loop(0, B, step, jnp.float32(0.0))
    out_ref[0, 0] = loss_sum


def _tc_bag_max(logits, zones, cats, labels):
    B, H, W = logits.shape
    grid_spec = pltpu.PrefetchScalarGridSpec(
        num_scalar_prefetch=2,
        grid=(),
        in_specs=[
            pl.BlockSpec(memory_space=pl.ANY),
            pl.BlockSpec(memory_space=pl.ANY),
        ],
        out_specs=pl.BlockSpec(memory_space=pltpu.SMEM),
        scratch_shapes=[
            pltpu.VMEM((_NSLOT, H, W), jnp.float32),
            pltpu.VMEM((_NSLOT, H, W), jnp.int32),
            pltpu.SMEM((1,), jnp.float32),
            pltpu.SemaphoreType.DMA((_NSLOT,)),
            pltpu.SemaphoreType.DMA((_NSLOT,)),
        ],
    )
    return pl.pallas_call(
        _tc_body,
        grid_spec=grid_spec,
        out_shape=jax.ShapeDtypeStruct((1, 1), jnp.float32),
    )(cats, labels, logits, zones)


def _bce_tail_body(max_ref, cats_ref, labels_ref, tcsum_ref, out_ref):
    x = jnp.max(max_ref[:, 0, :], axis=1, keepdims=True)
    c = cats_ref[:, 0, :]
    y = labels_ref[:, 0, :]
    valid = (c > 0) & (x > -9e29)
    r = jnp.where(valid, x, 0.0)
    per = jnp.maximum(r, 0.0) - r * y + jnp.log1p(jnp.exp(-jnp.abs(r)))
    out_ref[0, 0] = jnp.sum(per) + tcsum_ref[0, 0]


def kernel(pixel_logits, zone_patches, cats, labels):
    B, _, H, W = pixel_logits.shape
    N = H * W
    tc_b = B - _SC_B
    logits = pixel_logits.reshape(B, H, W)

    sc_logits = logits[tc_b:].reshape(_SC_B, N)
    sc_zones = zone_patches[tc_b:].reshape(_SC_B, N)
    sc_cats16 = jnp.broadcast_to(cats[tc_b:, None], (_SC_B, 16))
    sc_max = _sc_bag_max(_SC_B, N)(sc_logits, sc_zones, sc_cats16)

    tc_sum = _tc_bag_max(logits[:tc_b], zone_patches[:tc_b], cats[:tc_b],
                         labels[:tc_b])

    loss = pl.pallas_call(
        _bce_tail_body,
        in_specs=[
            pl.BlockSpec((_SC_B, 1, 32), lambda: (0, 0, 0)),
            pl.BlockSpec((_SC_B, 1, 1), lambda: (0, 0, 0)),
            pl.BlockSpec((_SC_B, 1, 1), lambda: (0, 0, 0)),
            pl.BlockSpec(memory_space=pltpu.SMEM),
        ],
        out_specs=pl.BlockSpec(memory_space=pltpu.SMEM),
        out_shape=jax.ShapeDtypeStruct((1, 1), jnp.float32),
    )(sc_max.reshape(_SC_B, 1, 32), cats[tc_b:].reshape(_SC_B, 1, 1),
      labels[tc_b:].reshape(_SC_B, 1, 1), tc_sum)

    return loss[0, 0] / B


# speedup vs baseline: 1.2588x; 1.0871x over previous
import functools

import jax
import jax.numpy as jnp
from jax import lax
from jax.experimental import pallas as pl
from jax.experimental.pallas import tpu as pltpu
from jax.experimental.pallas import tpu_sc as plsc

_NEG = -1e30
_NSLOT = 4
_CHUNK = 16384
_UNROLL = 16
_SC_B = 16


def _sc_bag_max(B0, B, N):
    nchunk = N // _CHUNK
    mesh = plsc.VectorSubcoreMesh(core_axis_name="c", subcore_axis_name="s")
    ncores, nsub = 2, 16
    nw = ncores * nsub

    @functools.partial(
        pl.kernel,
        mesh=mesh,
        out_type=jax.ShapeDtypeStruct((nw, 16), jnp.float32),
        scratch_types=[
            pltpu.VMEM((2, _CHUNK), jnp.float32),
            pltpu.VMEM((2, _CHUNK), jnp.int32),
            pltpu.VMEM((16,), jnp.int32),
            pltpu.VMEM((16,), jnp.float32),
            pltpu.SemaphoreType.DMA((2,)),
            pltpu.SemaphoreType.DMA((2,)),
        ],
    )
    def k(x_hbm, z_hbm, cats_hbm, out_hbm, xbuf, zbuf, cats_v, res_v, xsem, zsem):
        wid = lax.axis_index("s") * ncores + lax.axis_index("c")
        total = (B - B0) * nchunk
        cpw = total // nw
        base_g = wid * cpw
        my_b = B0 + base_g // nchunk

        def start(g, slot):
            b = B0 + g // nchunk
            off = (g % nchunk) * _CHUNK
            pltpu.async_copy(x_hbm.at[b, pl.ds(off, _CHUNK)], xbuf.at[slot],
                             xsem.at[slot])
            pltpu.async_copy(z_hbm.at[b, pl.ds(off, _CHUNK)], zbuf.at[slot],
                             zsem.at[slot])

        def wait(slot):
            pltpu.make_async_copy(x_hbm.at[0, pl.ds(0, _CHUNK)], xbuf.at[slot],
                                  xsem.at[slot]).wait()
            pltpu.make_async_copy(z_hbm.at[0, pl.ds(0, _CHUNK)], zbuf.at[slot],
                                  zsem.at[slot]).wait()

        start(base_g, 0)
        if cpw > 1:
            start(base_g + 1, 1)
        pltpu.sync_copy(cats_hbm.at[my_b], cats_v)
        cat = cats_v[...]

        acc = jnp.full((16,), _NEG, dtype=jnp.float32)
        for c in range(cpw):
            slot = c % 2
            wait(slot)

            def inner(i, acc):
                for u in range(_UNROLL):
                    base = i * (16 * _UNROLL) + u * 16
                    xv = xbuf[slot, pl.ds(base, 16)]
                    zv = zbuf[slot, pl.ds(base, 16)]
                    acc = jnp.where(zv == cat, jnp.maximum(acc, xv), acc)
                return acc

            acc = lax.fori_loop(0, _CHUNK // (16 * _UNROLL), inner, acc)

            if c + 2 < cpw:
                start(base_g + c + 2, slot)
        res_v[...] = acc
        pltpu.sync_copy(res_v, out_hbm.at[wid, :])

    return k


def _tc_body(nb, cats_ref, labels_ref, x_hbm, z_hbm, out_ref, xbuf, zbuf,
             acc_ref, xsem, zsem):
    B = nb

    def start(b, slot):
        pltpu.make_async_copy(x_hbm.at[b], xbuf.at[slot], xsem.at[slot]).start()
        pltpu.make_async_copy(z_hbm.at[b], zbuf.at[slot], zsem.at[slot]).start()

    for b in range(_NSLOT):
        start(b, b)

    def step(b, loss_sum):
        slot = jax.lax.rem(b, _NSLOT)
        pltpu.make_async_copy(x_hbm.at[0], xbuf.at[slot], xsem.at[slot]).wait()
        pltpu.make_async_copy(z_hbm.at[0], zbuf.at[slot], zsem.at[slot]).wait()
        x = xbuf[slot]
        z = zbuf[slot]
        cat = cats_ref[b]
        part = jnp.max(jnp.where(z == cat, x, _NEG))

        @pl.when(b + _NSLOT < B)
        def _next():
            start(b + _NSLOT, slot)

        valid = (cat > 0) & (part > -9e29)
        r = jnp.where(valid, part, 0.0)
        y = labels_ref[b]
        per = jnp.maximum(r, 0.0) - r * y + jnp.log1p(jnp.exp(-jnp.abs(r)))
        return loss_sum + per

    loss_sum = jax.lax.fori_loop(0, B, step, jnp.float32(0.0))
    out_ref[0, 0] = loss_sum


def _tc_bag_max(logits, zones, cats, labels, nb):
    B, H, W = logits.shape
    grid_spec = pltpu.PrefetchScalarGridSpec(
        num_scalar_prefetch=2,
        grid=(),
        in_specs=[
            pl.BlockSpec(memory_space=pl.ANY),
            pl.BlockSpec(memory_space=pl.ANY),
        ],
        out_specs=pl.BlockSpec(memory_space=pltpu.SMEM),
        scratch_shapes=[
            pltpu.VMEM((_NSLOT, H, W), jnp.float32),
            pltpu.VMEM((_NSLOT, H, W), jnp.int32),
            pltpu.SMEM((1,), jnp.float32),
            pltpu.SemaphoreType.DMA((_NSLOT,)),
            pltpu.SemaphoreType.DMA((_NSLOT,)),
        ],
    )
    return pl.pallas_call(
        functools.partial(_tc_body, nb),
        grid_spec=grid_spec,
        out_shape=jax.ShapeDtypeStruct((1, 1), jnp.float32),
    )(cats, labels, logits, zones)


def _bce_tail_body(max_ref, cats_ref, labels_ref, tcsum_ref, out_ref):
    x = jnp.max(max_ref[:, 0, :], axis=1, keepdims=True)
    c = cats_ref[-_SC_B:, 0, :]
    y = labels_ref[-_SC_B:, 0, :]
    valid = (c > 0) & (x > -9e29)
    r = jnp.where(valid, x, 0.0)
    per = jnp.maximum(r, 0.0) - r * y + jnp.log1p(jnp.exp(-jnp.abs(r)))
    out_ref[0, 0] = jnp.sum(per) + tcsum_ref[0, 0]


def kernel(pixel_logits, zone_patches, cats, labels):
    B, _, H, W = pixel_logits.shape
    N = H * W
    tc_b = B - _SC_B
    logits = pixel_logits.reshape(B, H, W)

    cats16 = jnp.broadcast_to(cats[:, None], (B, 16))
    sc_max = _sc_bag_max(tc_b, B, N)(
        pixel_logits.reshape(B, N), zone_patches.reshape(B, N), cats16)

    tc_sum = _tc_bag_max(logits, zone_patches, cats, labels, tc_b)

    loss = pl.pallas_call(
        _bce_tail_body,
        in_specs=[
            pl.BlockSpec((_SC_B, 1, 32), lambda: (0, 0, 0)),
            pl.BlockSpec((64, 1, 1), lambda: (0, 0, 0)),
            pl.BlockSpec((64, 1, 1), lambda: (0, 0, 0)),
            pl.BlockSpec(memory_space=pltpu.SMEM),
        ],
        out_specs=pl.BlockSpec(memory_space=pltpu.SMEM),
        out_shape=jax.ShapeDtypeStruct((1, 1), jnp.float32),
    )(sc_max.reshape(_SC_B, 1, 32), cats.reshape(B, 1, 1),
      labels.reshape(B, 1, 1), tc_sum)

    return loss[0, 0] / B


# speedup vs baseline: 3.1955x; 2.5386x over previous
import functools

import jax
import jax.numpy as jnp
from jax import lax
from jax.experimental import pallas as pl
from jax.experimental.pallas import tpu as pltpu
from jax.experimental.pallas import tpu_sc as plsc

_NEG = -1e30
_NSLOT = 4
_CROWS = 32
_UNROLL = 16
_SC_B = 16


def _sc_bag_max(B0, B, H, W):
    nchunk = H // _CROWS
    mesh = plsc.VectorSubcoreMesh(core_axis_name="c", subcore_axis_name="s")
    ncores, nsub = 2, 16
    nw = ncores * nsub

    @functools.partial(
        pl.kernel,
        mesh=mesh,
        out_type=jax.ShapeDtypeStruct((nw, 16), jnp.float32),
        scratch_types=[
            pltpu.VMEM((2, _CROWS, W), jnp.float32),
            pltpu.VMEM((2, _CROWS, W), jnp.int32),
            pltpu.VMEM((16,), jnp.int32),
            pltpu.VMEM((16,), jnp.float32),
            pltpu.SemaphoreType.DMA((2,)),
            pltpu.SemaphoreType.DMA((2,)),
        ],
    )
    def k(x_hbm, z_hbm, cats_hbm, out_hbm, xbuf, zbuf, cats_v, res_v, xsem, zsem):
        wid = lax.axis_index("s") * ncores + lax.axis_index("c")
        total = (B - B0) * nchunk
        cpw = total // nw
        base_g = wid * cpw
        my_b = B0 + base_g // nchunk

        def start(g, slot):
            b = B0 + g // nchunk
            r0 = (g % nchunk) * _CROWS
            pltpu.async_copy(x_hbm.at[b, pl.ds(r0, _CROWS), :], xbuf.at[slot],
                             xsem.at[slot])
            pltpu.async_copy(z_hbm.at[b, pl.ds(r0, _CROWS), :], zbuf.at[slot],
                             zsem.at[slot])

        def wait(slot):
            pltpu.make_async_copy(x_hbm.at[0, pl.ds(0, _CROWS), :],
                                  xbuf.at[slot], xsem.at[slot]).wait()
            pltpu.make_async_copy(z_hbm.at[0, pl.ds(0, _CROWS), :],
                                  zbuf.at[slot], zsem.at[slot]).wait()

        start(base_g, 0)
        if cpw > 1:
            start(base_g + 1, 1)
        pltpu.sync_copy(cats_hbm.at[my_b], cats_v)
        cat = cats_v[...]

        acc = jnp.full((16,), _NEG, dtype=jnp.float32)
        for c in range(cpw):
            slot = c % 2
            wait(slot)

            def inner(r, acc):
                for u in range(W // 16):
                    xv = xbuf[slot, r, pl.ds(u * 16, 16)]
                    zv = zbuf[slot, r, pl.ds(u * 16, 16)]
                    acc = jnp.where(zv == cat, jnp.maximum(acc, xv), acc)
                return acc

            acc = lax.fori_loop(0, _CROWS, inner, acc)

            if c + 2 < cpw:
                start(base_g + c + 2, slot)
        res_v[...] = acc
        pltpu.sync_copy(res_v, out_hbm.at[wid, :])

    return k


def _tc_body(nb, cats_ref, labels_ref, x_hbm, z_hbm, out_ref, xbuf, zbuf,
             acc_ref, xsem, zsem):
    B = nb

    def start(b, slot):
        pltpu.make_async_copy(x_hbm.at[b], xbuf.at[slot], xsem.at[slot]).start()
        pltpu.make_async_copy(z_hbm.at[b], zbuf.at[slot], zsem.at[slot]).start()

    for b in range(_NSLOT):
        start(b, b)

    def step(b, loss_sum):
        slot = jax.lax.rem(b, _NSLOT)
        pltpu.make_async_copy(x_hbm.at[0], xbuf.at[slot], xsem.at[slot]).wait()
        pltpu.make_async_copy(z_hbm.at[0], zbuf.at[slot], zsem.at[slot]).wait()
        x = xbuf[slot]
        z = zbuf[slot]
        cat = cats_ref[b]
        part = jnp.max(jnp.where(z == cat, x, _NEG))

        @pl.when(b + _NSLOT < B)
        def _next():
            start(b + _NSLOT, slot)

        valid = (cat > 0) & (part > -9e29)
        r = jnp.where(valid, part, 0.0)
        y = labels_ref[b]
        per = jnp.maximum(r, 0.0) - r * y + jnp.log1p(jnp.exp(-jnp.abs(r)))
        return loss_sum + per

    loss_sum = jax.lax.fori_loop(0, B, step, jnp.float32(0.0))
    out_ref[0, 0] = loss_sum


def _tc_bag_max(logits, zones, cats, labels, nb):
    B, H, W = logits.shape
    grid_spec = pltpu.PrefetchScalarGridSpec(
        num_scalar_prefetch=2,
        grid=(),
        in_specs=[
            pl.BlockSpec(memory_space=pl.ANY),
            pl.BlockSpec(memory_space=pl.ANY),
        ],
        out_specs=pl.BlockSpec(memory_space=pltpu.SMEM),
        scratch_shapes=[
            pltpu.VMEM((_NSLOT, H, W), jnp.float32),
            pltpu.VMEM((_NSLOT, H, W), jnp.int32),
            pltpu.SMEM((1,), jnp.float32),
            pltpu.SemaphoreType.DMA((_NSLOT,)),
            pltpu.SemaphoreType.DMA((_NSLOT,)),
        ],
    )
    return pl.pallas_call(
        functools.partial(_tc_body, nb),
        grid_spec=grid_spec,
        out_shape=jax.ShapeDtypeStruct((1, 1), jnp.float32),
    )(cats, labels, logits, zones)


def _bce_tail_body(max_ref, cats_ref, labels_ref, tcsum_ref, out_ref):
    x = jnp.max(max_ref[:, 0, :], axis=1, keepdims=True)
    c = cats_ref[-_SC_B:, 0, :]
    y = labels_ref[-_SC_B:, 0, :]
    valid = (c > 0) & (x > -9e29)
    r = jnp.where(valid, x, 0.0)
    per = jnp.maximum(r, 0.0) - r * y + jnp.log1p(jnp.exp(-jnp.abs(r)))
    out_ref[0, 0] = jnp.sum(per) + tcsum_ref[0, 0]


def kernel(pixel_logits, zone_patches, cats, labels):
    B, _, H, W = pixel_logits.shape
    N = H * W
    tc_b = B - _SC_B
    logits = pixel_logits.reshape(B, H, W)

    cats16 = jnp.broadcast_to(cats[:, None], (B, 16))
    sc_max = _sc_bag_max(tc_b, B, H, W)(logits, zone_patches, cats16)

    tc_sum = _tc_bag_max(logits, zone_patches, cats, labels, tc_b)

    loss = pl.pallas_call(
        _bce_tail_body,
        in_specs=[
            pl.BlockSpec((_SC_B, 1, 32), lambda: (0, 0, 0)),
            pl.BlockSpec((64, 1, 1), lambda: (0, 0, 0)),
            pl.BlockSpec((64, 1, 1), lambda: (0, 0, 0)),
            pl.BlockSpec(memory_space=pltpu.SMEM),
        ],
        out_specs=pl.BlockSpec(memory_space=pltpu.SMEM),
        out_shape=jax.ShapeDtypeStruct((1, 1), jnp.float32),
    )(sc_max.reshape(_SC_B, 1, 32), cats.reshape(B, 1, 1),
      labels.reshape(B, 1, 1), tc_sum)

    return loss[0, 0] / B


# speedup vs baseline: 5.0204x; 1.5711x over previous
import functools

import jax
import jax.numpy as jnp
from jax.experimental import pallas as pl
from jax.experimental.pallas import tpu as pltpu

_NEG = -1e30
_NSLOT = 4


def _body(cats_ref, labels_ref, x_hbm, z_hbm, out_ref, xbuf, zbuf, acc_ref,
          xsem, zsem):
    B = x_hbm.shape[0]

    def start(b, slot):
        pltpu.make_async_copy(x_hbm.at[b], xbuf.at[slot], xsem.at[slot]).start()
        pltpu.make_async_copy(z_hbm.at[b], zbuf.at[slot], zsem.at[slot]).start()

    for b in range(_NSLOT):
        start(b, b)

    def step(b, loss_sum):
        slot = jax.lax.rem(b, _NSLOT)
        pltpu.make_async_copy(x_hbm.at[0], xbuf.at[slot], xsem.at[slot]).wait()
        pltpu.make_async_copy(z_hbm.at[0], zbuf.at[slot], zsem.at[slot]).wait()
        x = xbuf[slot]
        z = zbuf[slot]
        cat = cats_ref[b]
        part = jnp.max(jnp.where(z == cat, x, _NEG))

        @pl.when(b + _NSLOT < B)
        def _next():
            start(b + _NSLOT, slot)

        valid = (cat > 0) & (part > -9e29)
        r = jnp.where(valid, part, 0.0)
        y = labels_ref[b]
        per = jnp.maximum(r, 0.0) - r * y + jnp.log1p(jnp.exp(-jnp.abs(r)))
        return loss_sum + per

    loss_sum = jax.lax.fori_loop(0, B, step, jnp.float32(0.0))
    out_ref[0, 0] = loss_sum / B


def kernel(pixel_logits, zone_patches, cats, labels):
    B, _, H, W = pixel_logits.shape
    logits = pixel_logits.reshape(B, H, W)

    grid_spec = pltpu.PrefetchScalarGridSpec(
        num_scalar_prefetch=2,
        grid=(),
        in_specs=[
            pl.BlockSpec(memory_space=pl.ANY),
            pl.BlockSpec(memory_space=pl.ANY),
        ],
        out_specs=pl.BlockSpec(memory_space=pltpu.SMEM),
        scratch_shapes=[
            pltpu.VMEM((_NSLOT, H, W), jnp.float32),
            pltpu.VMEM((_NSLOT, H, W), jnp.int32),
            pltpu.SMEM((1,), jnp.float32),
            pltpu.SemaphoreType.DMA((_NSLOT,)),
            pltpu.SemaphoreType.DMA((_NSLOT,)),
        ],
    )
    loss = pl.pallas_call(
        _body,
        grid_spec=grid_spec,
        out_shape=jax.ShapeDtypeStruct((1, 1), jnp.float32),
    )(cats, labels, logits, zone_patches)

    return loss[0, 0]
